# baseline (speedup 1.0000x reference)
import functools

import jax
import jax.numpy as jnp
from jax import lax
from jax.experimental import pallas as pl
from jax.experimental.pallas import tpu as pltpu
from jax.experimental.pallas import tpu_sc as plsc

_ALPHA = 0.1
_N = 16384
_C = 1000
_CP = 1024
_NT = 16
_CHUNK = _N // _NT
_BINS = 1024
_PER = _BINS // _NT

_SC_MESH = plsc.VectorSubcoreMesh(
    core_axis_name="c", subcore_axis_name="s", num_cores=1
)


@functools.partial(
    pl.kernel,
    out_type=jax.ShapeDtypeStruct((_BINS,), jnp.int32),
    mesh=_SC_MESH,
    scratch_types=[
        pltpu.VMEM((_CHUNK,), jnp.int32),
        pltpu.VMEM((_BINS,), jnp.int32),
        pltpu.VMEM_SHARED((_NT * _BINS,), jnp.int32),
        pltpu.VMEM((_NT * _BINS,), jnp.int32),
        pltpu.VMEM((_PER,), jnp.int32),
    ],
    compiler_params=pltpu.CompilerParams(needs_layout_passes=False),
)
def _sc_winner(target_hbm, win_hbm, tgt_v, bins_v, shared, allbins_v, win_v):
    s = lax.axis_index("s")
    base = s * _CHUNK
    pltpu.sync_copy(target_hbm.at[pl.ds(base, _CHUNK)], tgt_v)

    neg1 = jnp.full((16,), -1, jnp.int32)
    for k in range(_BINS // 16):
        bins_v[pl.ds(k * 16, 16)] = neg1

    lanes = lax.iota(jnp.int32, 16)

    def body(k, carry):
        tv = tgt_v[pl.ds(k * 16, 16)]
        vals = (base + k * 16) + lanes
        for j in range(16):
            plsc.store_scatter(bins_v, [tv], vals, mask=lanes == j)
        return carry

    lax.fori_loop(0, _CHUNK // 16, body, 0)

    pltpu.sync_copy(bins_v, shared.at[pl.ds(s * _BINS, _BINS)])
    plsc.subcore_barrier()
    pltpu.sync_copy(shared, allbins_v)

    cbase = s * _PER
    for c in range(_PER // 16):
        off = cbase + c * 16
        acc = allbins_v[pl.ds(off, 16)]
        for r in range(1, _NT):
            acc = jnp.maximum(acc, allbins_v[pl.ds(r * _BINS + off, 16)])
        win_v[pl.ds(c * 16, 16)] = acc

    pltpu.sync_copy(win_v, win_hbm.at[pl.ds(cbase, _PER)])


_NW_G = 32
_PER_G = _BINS // _NW_G


@functools.partial(
    pl.kernel,
    out_type=jax.ShapeDtypeStruct((_BINS, _CP), jnp.float32),
    mesh=plsc.VectorSubcoreMesh(core_axis_name="c", subcore_axis_name="s"),
    scratch_types=[
        pltpu.VMEM((_PER_G,), jnp.int32),
        pltpu.VMEM((_PER_G,), jnp.int32),
        pltpu.VMEM((_PER_G, _CP), jnp.float32),
        pltpu.SemaphoreType.DMA,
    ],
    compiler_params=pltpu.CompilerParams(needs_layout_passes=False),
)
def _sc_gather(win_hbm, padded_hbm, gth_hbm, win_v, idx_v, rows_v, sem):
    wid = lax.axis_index("s") * 2 + lax.axis_index("c")
    cbase = wid * _PER_G
    pltpu.sync_copy(win_hbm.at[pl.ds(cbase, _PER_G)], win_v)
    for c in range(_PER_G // 16):
        idx_v[pl.ds(c * 16, 16)] = jnp.maximum(win_v[pl.ds(c * 16, 16)], 0)
    pltpu.async_copy(padded_hbm.at[idx_v], rows_v, sem).wait()
    pltpu.sync_copy(rows_v, gth_hbm.at[pl.ds(cbase, _PER_G)])


_BC = 512


def _dense_body(xt_ref, f_ref, lse_ref, pad_ref):
    xt = xt_ref[...]
    m = jnp.max(xt, axis=0, keepdims=True)
    e = jnp.exp(xt - m)
    ones = jnp.ones((1, _C), jnp.float32)
    s = jnp.dot(ones, e, preferred_element_type=jnp.float32)
    u = e * e
    t = jnp.log(s - e)
    ut = u * t
    su = jnp.dot(ones, u, preferred_element_type=jnp.float32)
    sut = jnp.dot(ones, ut, preferred_element_type=jnp.float32)
    ls = jnp.log(s)
    r2 = 1.0 / (s * s)
    f_ref[...] = ((-_ALPHA) * (r2 * (sut - ls * su)))[0, :]
    lse_ref[...] = (m + ls)[0, :]
    pad_ref[...] = jnp.pad(xt.T, ((0, 0), (0, _CP - _C)))


_dense = pl.pallas_call(
    _dense_body,
    grid=(_N // _BC,),
    in_specs=[pl.BlockSpec((_C, _BC), lambda i: (0, i))],
    out_specs=[
        pl.BlockSpec((_BC,), lambda i: (i,)),
        pl.BlockSpec((_BC,), lambda i: (i,)),
        pl.BlockSpec((_BC, _CP), lambda i: (i, 0)),
    ],
    out_shape=[
        jax.ShapeDtypeStruct((_N,), jnp.float32),
        jax.ShapeDtypeStruct((_N,), jnp.float32),
        jax.ShapeDtypeStruct((_N, _CP), jnp.float32),
    ],
)


def _corr_body(xt_ref, g_ref, w_ref, f_ref, lse_ref, out_ref):
    xt = xt_ref[...]
    g = g_ref[...][:, :_C]
    w = w_ref[...]
    f = f_ref[...]
    lse = lse_ref[...]

    lse_h = lse[:_BINS]
    p2 = jnp.exp(2.0 * (xt - lse_h[None, :]))
    p2t = p2.T
    mg = jnp.max(g, axis=1)
    sg = jnp.sum(jnp.exp(g - mg[:, None]), axis=1)
    lse_g = mg + jnp.log(sg)
    ones = jnp.ones((_C, 1), jnp.float32)
    gdot = jnp.dot(p2t * g, ones, preferred_element_type=jnp.float32)[:, 0]
    s2 = jnp.dot(p2t, ones, preferred_element_type=jnp.float32)[:, 0]
    gterm = -(1.0 - _ALPHA) * (gdot - lse_g * s2)

    hit = w >= 0
    head = jnp.where(hit, gterm, f[:_BINS])
    out_ref[0, 0] = jnp.sum(head) + jnp.sum(f[_BINS:])


_corr = pl.pallas_call(
    _corr_body,
    grid=(1,),
    in_specs=[
        pl.BlockSpec((_C, _BINS), lambda i: (0, 0)),
        pl.BlockSpec((_BINS, _CP), lambda i: (0, 0)),
        pl.BlockSpec((_BINS,), lambda i: (0,)),
        pl.BlockSpec((_N,), lambda i: (0,)),
        pl.BlockSpec((_N,), lambda i: (0,)),
    ],
    out_specs=pl.BlockSpec((1, 1), lambda i: (0, 0), memory_space=pltpu.SMEM),
    out_shape=jax.ShapeDtypeStruct((1, 1), jnp.float32),
)


def kernel(pred, target):
    target = target.astype(jnp.int32)
    pred_t = pred.T
    win = _sc_winner(target)
    f, lse, padded = _dense(pred_t)
    gth = _sc_gather(win, padded)
    total = _corr(pred_t, gth, win, f, lse)
    return total[0, 0]


# speedup vs baseline: 1.0721x; 1.0721x over previous
import functools

import jax
import jax.numpy as jnp
from jax import lax
from jax.experimental import pallas as pl
from jax.experimental.pallas import tpu as pltpu
from jax.experimental.pallas import tpu_sc as plsc

_ALPHA = 0.1
_N = 16384
_C = 1000
_CP = 1024
_NT = 16
_CHUNK = _N // _NT
_BINS = 1024
_PER = _BINS // _NT

_SC_MESH = plsc.VectorSubcoreMesh(
    core_axis_name="c", subcore_axis_name="s", num_cores=1
)


@functools.partial(
    pl.kernel,
    out_type=jax.ShapeDtypeStruct((_BINS,), jnp.int32),
    mesh=_SC_MESH,
    scratch_types=[
        pltpu.VMEM((_CHUNK,), jnp.int32),
        pltpu.VMEM((_BINS,), jnp.int32),
        pltpu.VMEM_SHARED((_NT * _BINS,), jnp.int32),
        pltpu.VMEM((_NT * _BINS,), jnp.int32),
        pltpu.VMEM((_PER,), jnp.int32),
    ],
    compiler_params=pltpu.CompilerParams(needs_layout_passes=False),
)
def _sc_winner(target_hbm, win_hbm, tgt_v, bins_v, shared, allbins_v, win_v):
    s = lax.axis_index("s")
    base = s * _CHUNK
    pltpu.sync_copy(target_hbm.at[pl.ds(base, _CHUNK)], tgt_v)

    neg1 = jnp.full((16,), -1, jnp.int32)
    for k in range(_BINS // 16):
        bins_v[pl.ds(k * 16, 16)] = neg1

    lanes = lax.iota(jnp.int32, 16)

    def body(k, carry):
        tv = tgt_v[pl.ds(k * 16, 16)]
        vals = (base + k * 16) + lanes
        for j in range(16):
            plsc.store_scatter(bins_v, [tv], vals, mask=lanes == j)
        return carry

    lax.fori_loop(0, _CHUNK // 16, body, 0)

    pltpu.sync_copy(bins_v, shared.at[pl.ds(s * _BINS, _BINS)])
    plsc.subcore_barrier()
    pltpu.sync_copy(shared, allbins_v)

    cbase = s * _PER
    for c in range(_PER // 16):
        off = cbase + c * 16
        acc = allbins_v[pl.ds(off, 16)]
        for r in range(1, _NT):
            acc = jnp.maximum(acc, allbins_v[pl.ds(r * _BINS + off, 16)])
        win_v[pl.ds(c * 16, 16)] = acc

    pltpu.sync_copy(win_v, win_hbm.at[pl.ds(cbase, _PER)])


_CH = _CP // 2


@functools.partial(
    pl.kernel,
    out_type=jax.ShapeDtypeStruct((_BINS, _CH), jnp.int32),
    mesh=_SC_MESH,
    scratch_types=[
        pltpu.VMEM((_PER,), jnp.int32),
        pltpu.VMEM((_PER,), jnp.int32),
        pltpu.VMEM((_PER, _CH), jnp.int32),
        pltpu.SemaphoreType.DMA,
    ],
    compiler_params=pltpu.CompilerParams(needs_layout_passes=False),
)
def _sc_gather(win_hbm, padded_hbm, gth_hbm, win_v, idx_v, rows_v, sem):
    s = lax.axis_index("s")
    cbase = s * _PER
    pltpu.sync_copy(win_hbm.at[pl.ds(cbase, _PER)], win_v)
    for c in range(_PER // 16):
        idx_v[pl.ds(c * 16, 16)] = jnp.maximum(win_v[pl.ds(c * 16, 16)], 0)
    pltpu.async_copy(padded_hbm.at[idx_v], rows_v, sem).wait()
    pltpu.sync_copy(rows_v, gth_hbm.at[pl.ds(cbase, _PER)])


_BC = 512


def _dense_body(xt_ref, f_ref, lse_ref, pad_ref):
    xt = xt_ref[...]
    m = jnp.max(xt, axis=0, keepdims=True)
    e = jnp.exp(xt - m)
    ones = jnp.ones((1, _C), jnp.float32)
    s = jnp.dot(ones, e, preferred_element_type=jnp.float32)
    u = e * e
    t = jnp.log(s - e)
    ut = u * t
    su = jnp.dot(ones, u, preferred_element_type=jnp.float32)
    sut = jnp.dot(ones, ut, preferred_element_type=jnp.float32)
    ls = jnp.log(s)
    r2 = 1.0 / (s * s)
    f_ref[...] = ((-_ALPHA) * (r2 * (sut - ls * su)))[0, :]
    lse_ref[...] = (m + ls)[0, :]
    xp = jnp.pad(xt.T, ((0, 0), (0, _CP - _C)))
    u = lax.bitcast_convert_type(xp, jnp.uint32)
    b = (u + 0x7FFF + ((u >> 16) & 1)) >> 16
    pad_ref[...] = (b[:, :_CH] | (b[:, _CH:] << 16)).astype(jnp.int32)


_dense = pl.pallas_call(
    _dense_body,
    grid=(_N // _BC,),
    in_specs=[pl.BlockSpec((_C, _BC), lambda i: (0, i))],
    out_specs=[
        pl.BlockSpec((_BC,), lambda i: (i,)),
        pl.BlockSpec((_BC,), lambda i: (i,)),
        pl.BlockSpec((_BC, _CH), lambda i: (i, 0)),
    ],
    out_shape=[
        jax.ShapeDtypeStruct((_N,), jnp.float32),
        jax.ShapeDtypeStruct((_N,), jnp.float32),
        jax.ShapeDtypeStruct((_N, _CH), jnp.int32),
    ],
)


def _corr_body(xt_ref, g_ref, w_ref, f_ref, lse_ref, out_ref):
    xt = xt_ref[...]
    g32 = g_ref[...]
    lo = lax.bitcast_convert_type(g32 << 16, jnp.float32)
    hi = lax.bitcast_convert_type(g32 & jnp.int32(-65536), jnp.float32)
    g = jnp.concatenate([lo, hi], axis=1)[:, :_C]
    w = w_ref[...]
    f = f_ref[...]
    lse = lse_ref[...]

    lse_h = lse[:_BINS]
    p2 = jnp.exp(2.0 * (xt - lse_h[None, :]))
    p2t = p2.T
    mg = jnp.max(g, axis=1)
    sg = jnp.sum(jnp.exp(g - mg[:, None]), axis=1)
    lse_g = mg + jnp.log(sg)
    ones = jnp.ones((_C, 1), jnp.float32)
    gdot = jnp.dot(p2t * g, ones, preferred_element_type=jnp.float32)[:, 0]
    s2 = jnp.dot(p2t, ones, preferred_element_type=jnp.float32)[:, 0]
    gterm = -(1.0 - _ALPHA) * (gdot - lse_g * s2)

    hit = w >= 0
    head = jnp.where(hit, gterm, f[:_BINS])
    out_ref[0, 0] = jnp.sum(head) + jnp.sum(f[_BINS:])


_corr = pl.pallas_call(
    _corr_body,
    grid=(1,),
    in_specs=[
        pl.BlockSpec((_C, _BINS), lambda i: (0, 0)),
        pl.BlockSpec((_BINS, _CH), lambda i: (0, 0)),
        pl.BlockSpec((_BINS,), lambda i: (0,)),
        pl.BlockSpec((_N,), lambda i: (0,)),
        pl.BlockSpec((_N,), lambda i: (0,)),
    ],
    out_specs=pl.BlockSpec((1, 1), lambda i: (0, 0), memory_space=pltpu.SMEM),
    out_shape=jax.ShapeDtypeStruct((1, 1), jnp.float32),
)


def kernel(pred, target):
    target = target.astype(jnp.int32)
    pred_t = pred.T
    win = _sc_winner(target)
    f, lse, padded = _dense(pred_t)
    gth = _sc_gather(win, padded)
    total = _corr(pred_t, gth, win, f, lse)
    return total[0, 0]


# speedup vs baseline: 1.2019x; 1.1211x over previous
import functools

import jax
import jax.numpy as jnp
from jax import lax
from jax.experimental import pallas as pl
from jax.experimental.pallas import tpu as pltpu
from jax.experimental.pallas import tpu_sc as plsc

_ALPHA = 0.1
_N = 16384
_C = 1000
_CP = 1024
_NT = 16
_CHUNK = _N // _NT
_BINS = 1024
_PER = _BINS // _NT

_SC_MESH = plsc.VectorSubcoreMesh(
    core_axis_name="c", subcore_axis_name="s", num_cores=1
)


@functools.partial(
    pl.kernel,
    out_type=jax.ShapeDtypeStruct((_BINS,), jnp.int32),
    mesh=_SC_MESH,
    scratch_types=[
        pltpu.VMEM((_CHUNK,), jnp.int32),
        pltpu.VMEM((_BINS,), jnp.int32),
        pltpu.VMEM_SHARED((_NT * _BINS,), jnp.int32),
        pltpu.VMEM((_NT * _BINS,), jnp.int32),
        pltpu.VMEM((_PER,), jnp.int32),
    ],
    compiler_params=pltpu.CompilerParams(needs_layout_passes=False),
)
def _sc_winner(target_hbm, win_hbm, tgt_v, bins_v, shared, allbins_v, win_v):
    s = lax.axis_index("s")
    base = s * _CHUNK
    pltpu.sync_copy(target_hbm.at[pl.ds(base, _CHUNK)], tgt_v)

    neg1 = jnp.full((16,), -1, jnp.int32)
    for k in range(_BINS // 16):
        bins_v[pl.ds(k * 16, 16)] = neg1

    lanes = lax.iota(jnp.int32, 16)

    def body(k, carry):
        tv = tgt_v[pl.ds(k * 16, 16)]
        vals = (base + k * 16) + lanes
        for j in range(16):
            plsc.store_scatter(bins_v, [tv], vals, mask=lanes == j)
        return carry

    lax.fori_loop(0, _CHUNK // 16, body, 0)

    pltpu.sync_copy(bins_v, shared.at[pl.ds(s * _BINS, _BINS)])
    plsc.subcore_barrier()
    pltpu.sync_copy(shared, allbins_v)

    cbase = s * _PER
    for c in range(_PER // 16):
        off = cbase + c * 16
        acc = allbins_v[pl.ds(off, 16)]
        for r in range(1, _NT):
            acc = jnp.maximum(acc, allbins_v[pl.ds(r * _BINS + off, 16)])
        win_v[pl.ds(c * 16, 16)] = acc

    pltpu.sync_copy(win_v, win_hbm.at[pl.ds(cbase, _PER)])


_CH = _CP // 2


@functools.partial(
    pl.kernel,
    out_type=jax.ShapeDtypeStruct((_BINS, _CH), jnp.int32),
    mesh=_SC_MESH,
    scratch_types=[
        pltpu.VMEM((_PER,), jnp.int32),
        pltpu.VMEM((_PER,), jnp.int32),
        pltpu.VMEM((_PER, _CH), jnp.int32),
        pltpu.SemaphoreType.DMA,
    ],
    compiler_params=pltpu.CompilerParams(needs_layout_passes=False),
)
def _sc_gather(win_hbm, padded_hbm, gth_hbm, win_v, idx_v, rows_v, sem):
    s = lax.axis_index("s")
    cbase = s * _PER
    pltpu.sync_copy(win_hbm.at[pl.ds(cbase, _PER)], win_v)
    for c in range(_PER // 16):
        idx_v[pl.ds(c * 16, 16)] = jnp.maximum(win_v[pl.ds(c * 16, 16)], 0)
    pltpu.async_copy(padded_hbm.at[idx_v], rows_v, sem).wait()
    pltpu.sync_copy(rows_v, gth_hbm.at[pl.ds(cbase, _PER)])


_BC = 1024


def _dense_body(xt_ref, f_ref, lse_ref, pad_ref):
    xt = xt_ref[...]
    m = jnp.max(xt, axis=0, keepdims=True)
    e = jnp.exp(xt - m)
    ones = jnp.ones((1, _C), jnp.float32)
    s = jnp.dot(ones, e, preferred_element_type=jnp.float32)
    u = e * e
    t = jnp.log(s - e)
    ut = u * t
    su = jnp.dot(ones, u, preferred_element_type=jnp.float32)
    sut = jnp.dot(ones, ut, preferred_element_type=jnp.float32)
    ls = jnp.log(s)
    r2 = 1.0 / (s * s)
    f_ref[...] = ((-_ALPHA) * (r2 * (sut - ls * su)))[0, :]
    lse_ref[...] = (m + ls)[0, :]
    xp = jnp.pad(xt.T, ((0, 0), (0, _CP - _C)))
    u = lax.bitcast_convert_type(xp, jnp.uint32)
    b = (u + 0x7FFF + ((u >> 16) & 1)) >> 16
    pad_ref[...] = (b[:, :_CH] | (b[:, _CH:] << 16)).astype(jnp.int32)


_dense = pl.pallas_call(
    _dense_body,
    grid=(_N // _BC,),
    in_specs=[pl.BlockSpec((_C, _BC), lambda i: (0, i))],
    out_specs=[
        pl.BlockSpec((_BC,), lambda i: (i,)),
        pl.BlockSpec((_BC,), lambda i: (i,)),
        pl.BlockSpec((_BC, _CH), lambda i: (i, 0)),
    ],
    out_shape=[
        jax.ShapeDtypeStruct((_N,), jnp.float32),
        jax.ShapeDtypeStruct((_N,), jnp.float32),
        jax.ShapeDtypeStruct((_N, _CH), jnp.int32),
    ],
)


def _corr_body(xt_ref, g_ref, w_ref, f_ref, lse_ref, out_ref):
    xt = xt_ref[...]
    g32 = g_ref[...]
    lo = lax.bitcast_convert_type(g32 << 16, jnp.float32)
    hi = lax.bitcast_convert_type(g32 & jnp.int32(-65536), jnp.float32)
    g = jnp.concatenate([lo, hi], axis=1)[:, :_C]
    w = w_ref[...]
    f = f_ref[...]
    lse = lse_ref[...]

    lse_h = lse[:_BINS]
    p2 = jnp.exp(2.0 * (xt - lse_h[None, :]))
    p2t = p2.T
    mg = jnp.max(g, axis=1)
    sg = jnp.sum(jnp.exp(g - mg[:, None]), axis=1)
    lse_g = mg + jnp.log(sg)
    ones = jnp.ones((_C, 1), jnp.float32)
    gdot = jnp.dot(p2t * g, ones, preferred_element_type=jnp.float32)[:, 0]
    s2 = jnp.dot(p2t, ones, preferred_element_type=jnp.float32)[:, 0]
    gterm = -(1.0 - _ALPHA) * (gdot - lse_g * s2)

    hit = w >= 0
    head = jnp.where(hit, gterm, f[:_BINS])
    out_ref[0, 0] = jnp.sum(head) + jnp.sum(f[_BINS:])


_corr = pl.pallas_call(
    _corr_body,
    grid=(1,),
    in_specs=[
        pl.BlockSpec((_C, _BINS), lambda i: (0, 0)),
        pl.BlockSpec((_BINS, _CH), lambda i: (0, 0)),
        pl.BlockSpec((_BINS,), lambda i: (0,)),
        pl.BlockSpec((_N,), lambda i: (0,)),
        pl.BlockSpec((_N,), lambda i: (0,)),
    ],
    out_specs=pl.BlockSpec((1, 1), lambda i: (0, 0), memory_space=pltpu.SMEM),
    out_shape=jax.ShapeDtypeStruct((1, 1), jnp.float32),
)


def kernel(pred, target):
    target = target.astype(jnp.int32)
    pred_t = pred.T
    win = _sc_winner(target)
    f, lse, padded = _dense(pred_t)
    gth = _sc_gather(win, padded)
    total = _corr(pred_t, gth, win, f, lse)
    return total[0, 0]


# speedup vs baseline: 1.2477x; 1.0381x over previous
import functools

import jax
import jax.numpy as jnp
from jax import lax
from jax.experimental import pallas as pl
from jax.experimental.pallas import tpu as pltpu
from jax.experimental.pallas import tpu_sc as plsc

_ALPHA = 0.1
_N = 16384
_C = 1000
_CP = 1024
_NT = 16
_CHUNK = _N // _NT
_BINS = 1024
_PER = _BINS // _NT

_SC_MESH = plsc.VectorSubcoreMesh(
    core_axis_name="c", subcore_axis_name="s", num_cores=1
)


@functools.partial(
    pl.kernel,
    out_type=jax.ShapeDtypeStruct((_BINS,), jnp.int32),
    mesh=_SC_MESH,
    scratch_types=[
        pltpu.VMEM((_CHUNK,), jnp.int32),
        pltpu.VMEM((_BINS,), jnp.int32),
        pltpu.VMEM_SHARED((_NT * _BINS,), jnp.int32),
        pltpu.VMEM((_NT * _BINS,), jnp.int32),
        pltpu.VMEM((_PER,), jnp.int32),
    ],
    compiler_params=pltpu.CompilerParams(needs_layout_passes=False),
)
def _sc_winner(target_hbm, win_hbm, tgt_v, bins_v, shared, allbins_v, win_v):
    s = lax.axis_index("s")
    base = s * _CHUNK
    pltpu.sync_copy(target_hbm.at[pl.ds(base, _CHUNK)], tgt_v)

    neg1 = jnp.full((16,), -1, jnp.int32)
    for k in range(_BINS // 16):
        bins_v[pl.ds(k * 16, 16)] = neg1

    lanes = lax.iota(jnp.int32, 16)

    def body(k, carry):
        tv = tgt_v[pl.ds(k * 16, 16)]
        vals = (base + k * 16) + lanes
        for j in range(16):
            plsc.store_scatter(bins_v, [tv], vals, mask=lanes == j)
        return carry

    lax.fori_loop(0, _CHUNK // 16, body, 0)

    pltpu.sync_copy(bins_v, shared.at[pl.ds(s * _BINS, _BINS)])
    plsc.subcore_barrier()
    pltpu.sync_copy(shared, allbins_v)

    cbase = s * _PER
    for c in range(_PER // 16):
        off = cbase + c * 16
        acc = allbins_v[pl.ds(off, 16)]
        for r in range(1, _NT):
            acc = jnp.maximum(acc, allbins_v[pl.ds(r * _BINS + off, 16)])
        win_v[pl.ds(c * 16, 16)] = acc

    pltpu.sync_copy(win_v, win_hbm.at[pl.ds(cbase, _PER)])


_CH = _CP // 2


@functools.partial(
    pl.kernel,
    out_type=jax.ShapeDtypeStruct((_BINS, _CH), jnp.int32),
    mesh=_SC_MESH,
    scratch_types=[
        pltpu.VMEM((_PER,), jnp.int32),
        pltpu.VMEM((_PER,), jnp.int32),
        pltpu.VMEM((_PER, _CH), jnp.int32),
        pltpu.SemaphoreType.DMA,
    ],
    compiler_params=pltpu.CompilerParams(needs_layout_passes=False),
)
def _sc_gather(win_hbm, padded_hbm, gth_hbm, win_v, idx_v, rows_v, sem):
    s = lax.axis_index("s")
    cbase = s * _PER
    pltpu.sync_copy(win_hbm.at[pl.ds(cbase, _PER)], win_v)
    for c in range(_PER // 16):
        idx_v[pl.ds(c * 16, 16)] = jnp.maximum(win_v[pl.ds(c * 16, 16)], 0)
    pltpu.async_copy(padded_hbm.at[idx_v], rows_v, sem).wait()
    pltpu.sync_copy(rows_v, gth_hbm.at[pl.ds(cbase, _PER)])


_BC = 2048


def _dense_body(xt_ref, f_ref, lse_ref, pad_ref):
    xt = xt_ref[...]
    m = jnp.max(xt, axis=0, keepdims=True)
    e = jnp.exp(xt - m)
    ones = jnp.ones((1, _C), jnp.float32)
    s = jnp.dot(ones, e, preferred_element_type=jnp.float32)
    u = e * e
    t = jnp.log(s - e)
    ut = u * t
    su = jnp.dot(ones, u, preferred_element_type=jnp.float32)
    sut = jnp.dot(ones, ut, preferred_element_type=jnp.float32)
    ls = jnp.log(s)
    r2 = 1.0 / (s * s)
    f_ref[...] = ((-_ALPHA) * (r2 * (sut - ls * su)))[0, :]
    lse_ref[...] = (m + ls)[0, :]
    xp = jnp.pad(xt.T, ((0, 0), (0, _CP - _C)))
    u = lax.bitcast_convert_type(xp, jnp.uint32)
    b = (u + 0x7FFF + ((u >> 16) & 1)) >> 16
    pad_ref[...] = (b[:, :_CH] | (b[:, _CH:] << 16)).astype(jnp.int32)


_dense = pl.pallas_call(
    _dense_body,
    grid=(_N // _BC,),
    in_specs=[pl.BlockSpec((_C, _BC), lambda i: (0, i))],
    out_specs=[
        pl.BlockSpec((_BC,), lambda i: (i,)),
        pl.BlockSpec((_BC,), lambda i: (i,)),
        pl.BlockSpec((_BC, _CH), lambda i: (i, 0)),
    ],
    out_shape=[
        jax.ShapeDtypeStruct((_N,), jnp.float32),
        jax.ShapeDtypeStruct((_N,), jnp.float32),
        jax.ShapeDtypeStruct((_N, _CH), jnp.int32),
    ],
)


def _corr_body(xt_ref, g_ref, w_ref, f_ref, lse_ref, out_ref):
    xt = xt_ref[...]
    g32 = g_ref[...]
    lo = lax.bitcast_convert_type(g32 << 16, jnp.float32)
    hi = lax.bitcast_convert_type(g32 & jnp.int32(-65536), jnp.float32)
    g = jnp.concatenate([lo, hi], axis=1)[:, :_C]
    w = w_ref[...]
    f = f_ref[...]
    lse = lse_ref[...]

    lse_h = lse[:_BINS]
    p2 = jnp.exp(2.0 * (xt - lse_h[None, :]))
    p2t = p2.T
    mg = jnp.max(g, axis=1)
    sg = jnp.sum(jnp.exp(g - mg[:, None]), axis=1)
    lse_g = mg + jnp.log(sg)
    ones = jnp.ones((_C, 1), jnp.float32)
    gdot = jnp.dot(p2t * g, ones, preferred_element_type=jnp.float32)[:, 0]
    s2 = jnp.dot(p2t, ones, preferred_element_type=jnp.float32)[:, 0]
    gterm = -(1.0 - _ALPHA) * (gdot - lse_g * s2)

    hit = w >= 0
    head = jnp.where(hit, gterm, f[:_BINS])
    out_ref[0, 0] = jnp.sum(head) + jnp.sum(f[_BINS:])


_corr = pl.pallas_call(
    _corr_body,
    grid=(1,),
    in_specs=[
        pl.BlockSpec((_C, _BINS), lambda i: (0, 0)),
        pl.BlockSpec((_BINS, _CH), lambda i: (0, 0)),
        pl.BlockSpec((_BINS,), lambda i: (0,)),
        pl.BlockSpec((_N,), lambda i: (0,)),
        pl.BlockSpec((_N,), lambda i: (0,)),
    ],
    out_specs=pl.BlockSpec((1, 1), lambda i: (0, 0), memory_space=pltpu.SMEM),
    out_shape=jax.ShapeDtypeStruct((1, 1), jnp.float32),
)


def kernel(pred, target):
    target = target.astype(jnp.int32)
    pred_t = pred.T
    win = _sc_winner(target)
    f, lse, padded = _dense(pred_t)
    gth = _sc_gather(win, padded)
    total = _corr(pred_t, gth, win, f, lse)
    return total[0, 0]
